# baseline (speedup 1.0000x reference)
import functools

import jax
import jax.numpy as jnp
from jax import lax
from jax.experimental import pallas as pl
from jax.experimental.pallas import tpu as pltpu
from jax.experimental.pallas import tpu_sc as plsc

N_RAY = 8192
N_SAMP = 192
FLAT = N_SAMP * 3
NW = 32
RPW = N_RAY // NW
G = 16
NG = RPW // G
f32 = jnp.float32
i32 = jnp.int32


def _sc_body(rgb_h, nbs_h, sig_h, z_h,
             comp_h, w_h, d_h, o_h,
             rgb_v, nb_v, sig_v, z_v, w_v, comp_s, d_s, o_s):
    wid = lax.axis_index("s") * 2 + lax.axis_index("c")
    iota = lax.iota(i32, 16)

    def group_body(g, _):
        r0 = wid * RPW + g * G
        pltpu.sync_copy(rgb_h.at[pl.ds(r0, G)], rgb_v)
        for i in range(5):
            pltpu.sync_copy(nbs_h.at[pl.ds(i * N_RAY + r0, G)],
                            nb_v.at[pl.ds(i * G, G)])
        pltpu.sync_copy(sig_h.at[pl.ds(r0, G)], sig_v)
        pltpu.sync_copy(z_h.at[pl.ds(r0, G)], z_v)

        def samp_body(s, carry):
            tr, zz, ar, ag, ab, ad, ao = carry
            vs = jnp.full((16,), s, i32)
            sig = plsc.load_gather(sig_v, [iota, vs])
            vs1 = jnp.full((16,), jnp.minimum(s + 1, N_SAMP - 1), i32)
            zn = plsc.load_gather(z_v, [iota, vs1])
            delta = jnp.where(s == N_SAMP - 1, jnp.full((16,), 1e10, f32),
                              zn - zz)
            ds = delta * jnp.maximum(sig, 0.0)
            tr_next = tr * jnp.exp(-ds)
            w = tr - tr_next
            plsc.store_scatter(w_v, [iota, vs], w)
            m = w >= 0.01
            col0 = jnp.full((16,), s * 3, i32)
            col1 = jnp.full((16,), s * 3 + 1, i32)
            col2 = jnp.full((16,), s * 3 + 2, i32)
            rc0 = plsc.load_gather(rgb_v, [iota, col0])
            rc1 = plsc.load_gather(rgb_v, [iota, col1])
            rc2 = plsc.load_gather(rgb_v, [iota, col2])
            den = jnp.full((16,), 1.0, f32)
            s0 = jnp.zeros((16,), f32)
            s1 = jnp.zeros((16,), f32)
            s2 = jnp.zeros((16,), f32)
            for i in range(5):
                row = iota + i * G
                n0 = plsc.load_gather(nb_v, [row, col0])
                n1 = plsc.load_gather(nb_v, [row, col1])
                n2 = plsc.load_gather(nb_v, [row, col2])
                d0 = rc0 - n0
                d1 = rc1 - n1
                d2 = rc2 - n2
                wi = jnp.exp(-(d0 * d0 + d1 * d1 + d2 * d2))
                den = den + wi
                s0 = s0 + n0 * wi
                s1 = s1 + n1 * wi
                s2 = s2 + n2 * wi
            inv = 1.0 / den
            rn0 = jnp.where(m, (rc0 + s0) * inv, rc0)
            rn1 = jnp.where(m, (rc1 + s1) * inv, rc1)
            rn2 = jnp.where(m, (rc2 + s2) * inv, rc2)
            return (tr_next, zn, ar + w * rn0, ag + w * rn1, ab + w * rn2,
                    ad + w * zz, ao + w)

        zv16 = jnp.zeros((16,), f32)
        one16 = jnp.full((16,), 1.0, f32)
        z0 = plsc.load_gather(z_v, [iota, jnp.zeros((16,), i32)])
        _, _, ar, ag, ab, ad, ao = lax.fori_loop(
            0, N_SAMP, samp_body,
            (one16, z0, zv16, zv16, zv16, zv16, zv16))

        plsc.store_scatter(comp_s, [iota, jnp.zeros((16,), i32)], ar)
        plsc.store_scatter(comp_s, [iota, jnp.full((16,), 1, i32)], ag)
        plsc.store_scatter(comp_s, [iota, jnp.full((16,), 2, i32)], ab)
        d_s[...] = ad
        o_s[...] = ao
        pltpu.sync_copy(w_v, w_h.at[pl.ds(r0, G)])
        pltpu.sync_copy(comp_s, comp_h.at[pl.ds(r0, G)])
        pltpu.sync_copy(d_s, d_h.at[pl.ds(r0, G)])
        pltpu.sync_copy(o_s, o_h.at[pl.ds(r0, G)])
        return 0

    lax.fori_loop(0, NG, group_body, 0)


@jax.jit
def _run_sc(rgb2, nbs2, sigma, z_vals):
    fn = functools.partial(
        pl.kernel,
        out_type=[
            jax.ShapeDtypeStruct((N_RAY, 3), f32),
            jax.ShapeDtypeStruct((N_RAY, N_SAMP), f32),
            jax.ShapeDtypeStruct((N_RAY,), f32),
            jax.ShapeDtypeStruct((N_RAY,), f32),
        ],
        mesh=plsc.VectorSubcoreMesh(core_axis_name="c", subcore_axis_name="s"),
        compiler_params=pltpu.CompilerParams(
            needs_layout_passes=False, use_tc_tiling_on_sc=False),
        scratch_types=[
            pltpu.VMEM((G, FLAT), f32),
            pltpu.VMEM((5 * G, FLAT), f32),
            pltpu.VMEM((G, N_SAMP), f32),
            pltpu.VMEM((G, N_SAMP), f32),
            pltpu.VMEM((G, N_SAMP), f32),
            pltpu.VMEM((G, 3), f32),
            pltpu.VMEM((16,), f32),
            pltpu.VMEM((16,), f32),
        ],
    )(_sc_body)
    return fn(rgb2, nbs2, sigma, z_vals)


def kernel(rgb, neighbor_rgbs, sigma, z_vals, white_bkgd):
    comp, w, depth, opac = _run_sc(
        rgb.reshape(N_RAY, FLAT), neighbor_rgbs.reshape(5 * N_RAY, FLAT),
        sigma, z_vals)
    comp_rgb = jnp.where(white_bkgd, comp + (1.0 - opac)[:, None], comp)
    return comp_rgb, depth, opac, w


# speedup vs baseline: 1.1316x; 1.1316x over previous
import functools

import jax
import jax.numpy as jnp
from jax import lax
from jax.experimental import pallas as pl
from jax.experimental.pallas import tpu as pltpu
from jax.experimental.pallas import tpu_sc as plsc

N_RAY = 8192
N_SAMP = 192
FLAT = N_SAMP * 3
NW = 32
RPW = N_RAY // NW
G = 16
NG = RPW // G
f32 = jnp.float32
i32 = jnp.int32


def _sc_body(rgb_h, nbs_h, sig_h, z_h,
             comp_h, w_h, d_h, o_h,
             rgb_a, nb_a, sig_a, z_a, rgb_b, nb_b, sig_b, z_b,
             w_v, comp_s, d_s, o_s, sem_a, sem_b):
    wid = lax.axis_index("s") * 2 + lax.axis_index("c")
    iota = lax.iota(i32, 16)

    def copies(g, rgb_v, nb_v, sig_v, z_v, sem):
        r0 = wid * RPW + g * G
        yield rgb_h.at[pl.ds(r0, G)], rgb_v, sem
        for i in range(5):
            yield (nbs_h.at[pl.ds(i * N_RAY + r0, G)],
                   nb_v.at[pl.ds(i * G, G)], sem)
        yield sig_h.at[pl.ds(r0, G)], sig_v, sem
        yield z_h.at[pl.ds(r0, G)], z_v, sem

    def fire(g, rgb_v, nb_v, sig_v, z_v, sem):
        for src, dst, sm in copies(g, rgb_v, nb_v, sig_v, z_v, sem):
            pltpu.async_copy(src, dst, sm)

    def drain(g, rgb_v, nb_v, sig_v, z_v, sem):
        for src, dst, sm in copies(g, rgb_v, nb_v, sig_v, z_v, sem):
            pltpu.make_async_copy(src, dst, sm).wait()

    def compute(g, rgb_v, nb_v, sig_v, z_v):
        r0 = wid * RPW + g * G

        def samp_body(s, carry):
            tr, zz, ar, ag, ab, ad, ao = carry
            vs = jnp.full((16,), s, i32)
            sig = plsc.load_gather(sig_v, [iota, vs])
            vs1 = jnp.full((16,), jnp.minimum(s + 1, N_SAMP - 1), i32)
            zn = plsc.load_gather(z_v, [iota, vs1])
            delta = jnp.where(s == N_SAMP - 1, jnp.full((16,), 1e10, f32),
                              zn - zz)
            ds = delta * jnp.maximum(sig, 0.0)
            tr_next = tr * jnp.exp(-ds)
            w = tr - tr_next
            plsc.store_scatter(w_v, [iota, vs], w)
            m = w >= 0.01
            col0 = jnp.full((16,), s * 3, i32)
            col1 = jnp.full((16,), s * 3 + 1, i32)
            col2 = jnp.full((16,), s * 3 + 2, i32)
            rc0 = plsc.load_gather(rgb_v, [iota, col0])
            rc1 = plsc.load_gather(rgb_v, [iota, col1])
            rc2 = plsc.load_gather(rgb_v, [iota, col2])
            den = jnp.full((16,), 1.0, f32)
            s0 = jnp.zeros((16,), f32)
            s1 = jnp.zeros((16,), f32)
            s2 = jnp.zeros((16,), f32)
            for i in range(5):
                row = iota + i * G
                n0 = plsc.load_gather(nb_v, [row, col0])
                n1 = plsc.load_gather(nb_v, [row, col1])
                n2 = plsc.load_gather(nb_v, [row, col2])
                d0 = rc0 - n0
                d1 = rc1 - n1
                d2 = rc2 - n2
                wi = jnp.exp(-(d0 * d0 + d1 * d1 + d2 * d2))
                den = den + wi
                s0 = s0 + n0 * wi
                s1 = s1 + n1 * wi
                s2 = s2 + n2 * wi
            inv = 1.0 / den
            rn0 = jnp.where(m, (rc0 + s0) * inv, rc0)
            rn1 = jnp.where(m, (rc1 + s1) * inv, rc1)
            rn2 = jnp.where(m, (rc2 + s2) * inv, rc2)
            return (tr_next, zn, ar + w * rn0, ag + w * rn1, ab + w * rn2,
                    ad + w * zz, ao + w)

        zv16 = jnp.zeros((16,), f32)
        one16 = jnp.full((16,), 1.0, f32)
        z0 = plsc.load_gather(z_v, [iota, jnp.zeros((16,), i32)])
        _, _, ar, ag, ab, ad, ao = lax.fori_loop(
            0, N_SAMP, samp_body,
            (one16, z0, zv16, zv16, zv16, zv16, zv16))

        plsc.store_scatter(comp_s, [iota, jnp.zeros((16,), i32)], ar)
        plsc.store_scatter(comp_s, [iota, jnp.full((16,), 1, i32)], ag)
        plsc.store_scatter(comp_s, [iota, jnp.full((16,), 2, i32)], ab)
        d_s[...] = ad
        o_s[...] = ao
        pltpu.sync_copy(w_v, w_h.at[pl.ds(r0, G)])
        pltpu.sync_copy(comp_s, comp_h.at[pl.ds(r0, G)])
        pltpu.sync_copy(d_s, d_h.at[pl.ds(r0, G)])
        pltpu.sync_copy(o_s, o_h.at[pl.ds(r0, G)])

    bufs_a = (rgb_a, nb_a, sig_a, z_a)
    bufs_b = (rgb_b, nb_b, sig_b, z_b)
    fire(0, *bufs_a, sem_a)

    def pair_body(k, _):
        g0 = 2 * k
        drain(g0, *bufs_a, sem_a)
        fire(g0 + 1, *bufs_b, sem_b)
        compute(g0, *bufs_a)
        drain(g0 + 1, *bufs_b, sem_b)

        @pl.when(k < NG // 2 - 1)
        def _():
            fire(g0 + 2, *bufs_a, sem_a)

        compute(g0 + 1, *bufs_b)
        return 0

    lax.fori_loop(0, NG // 2, pair_body, 0)


@jax.jit
def _run_sc(rgb2, nbs2, sigma, z_vals):
    fn = functools.partial(
        pl.kernel,
        out_type=[
            jax.ShapeDtypeStruct((N_RAY, 3), f32),
            jax.ShapeDtypeStruct((N_RAY, N_SAMP), f32),
            jax.ShapeDtypeStruct((N_RAY,), f32),
            jax.ShapeDtypeStruct((N_RAY,), f32),
        ],
        mesh=plsc.VectorSubcoreMesh(core_axis_name="c", subcore_axis_name="s"),
        compiler_params=pltpu.CompilerParams(
            needs_layout_passes=False, use_tc_tiling_on_sc=False),
        scratch_types=[
            pltpu.VMEM((G, FLAT), f32),
            pltpu.VMEM((5 * G, FLAT), f32),
            pltpu.VMEM((G, N_SAMP), f32),
            pltpu.VMEM((G, N_SAMP), f32),
            pltpu.VMEM((G, FLAT), f32),
            pltpu.VMEM((5 * G, FLAT), f32),
            pltpu.VMEM((G, N_SAMP), f32),
            pltpu.VMEM((G, N_SAMP), f32),
            pltpu.VMEM((G, N_SAMP), f32),
            pltpu.VMEM((G, 3), f32),
            pltpu.VMEM((16,), f32),
            pltpu.VMEM((16,), f32),
            pltpu.SemaphoreType.DMA,
            pltpu.SemaphoreType.DMA,
        ],
    )(_sc_body)
    return fn(rgb2, nbs2, sigma, z_vals)


def kernel(rgb, neighbor_rgbs, sigma, z_vals, white_bkgd):
    comp, w, depth, opac = _run_sc(
        rgb.reshape(N_RAY, FLAT), neighbor_rgbs.reshape(5 * N_RAY, FLAT),
        sigma, z_vals)
    comp_rgb = jnp.where(white_bkgd, comp + (1.0 - opac)[:, None], comp)
    return comp_rgb, depth, opac, w


# speedup vs baseline: 1.6610x; 1.4678x over previous
import functools

import jax
import jax.numpy as jnp
from jax import lax
from jax.experimental import pallas as pl
from jax.experimental.pallas import tpu as pltpu
from jax.experimental.pallas import tpu_sc as plsc

N_RAY = 8192
N_SAMP = 192
FLAT = N_SAMP * 3
NW = 32
RPW = N_RAY // NW
G = 16
NG = RPW // G
f32 = jnp.float32
i32 = jnp.int32


def _sc_body(rgb_h, nbs_h, sig_h, z_h,
             comp_h, w_h, d_h, o_h,
             rgb_a, nb_a, sig_a, z_a, rgb_b, nb_b, sig_b, z_b,
             w_v, comp_s, d_s, o_s, sem_a, sem_b):
    wid = lax.axis_index("s") * 2 + lax.axis_index("c")
    iota = lax.iota(i32, 16)

    def copies(g, rgb_v, nb_v, sig_v, z_v, sem):
        r0 = wid * RPW + g * G
        yield rgb_h.at[pl.ds(r0, G)], rgb_v.at[:, pl.ds(0, FLAT)], sem
        for i in range(5):
            yield (nbs_h.at[pl.ds(i * N_RAY + r0, G)],
                   nb_v.at[pl.ds(i * G, G), pl.ds(0, FLAT)], sem)
        yield sig_h.at[pl.ds(r0, G)], sig_v.at[:, pl.ds(0, N_SAMP)], sem
        yield z_h.at[pl.ds(r0, G)], z_v.at[:, pl.ds(0, N_SAMP)], sem

    def fire(g, rgb_v, nb_v, sig_v, z_v, sem):
        for src, dst, sm in copies(g, rgb_v, nb_v, sig_v, z_v, sem):
            pltpu.async_copy(src, dst, sm)

    def drain(g, rgb_v, nb_v, sig_v, z_v, sem):
        for src, dst, sm in copies(g, rgb_v, nb_v, sig_v, z_v, sem):
            pltpu.make_async_copy(src, dst, sm).wait()

    def compute(g, rgb_v, nb_v, sig_v, z_v):
        r0 = wid * RPW + g * G

        def samp_body(s, carry):
            tr, zz, ar, ag, ab, ad, ao = carry
            vs = jnp.full((16,), s, i32)
            sig = plsc.load_gather(sig_v, [iota, vs])
            vs1 = jnp.full((16,), jnp.minimum(s + 1, N_SAMP - 1), i32)
            zn = plsc.load_gather(z_v, [iota, vs1])
            delta = jnp.where(s == N_SAMP - 1, jnp.full((16,), 1e10, f32),
                              zn - zz)
            ds = delta * jnp.maximum(sig, 0.0)
            tr_next = tr * jnp.exp(-ds)
            w = tr - tr_next
            plsc.store_scatter(w_v, [iota, vs], w)
            m = w >= 0.01
            col0 = jnp.full((16,), s * 3, i32)
            col1 = jnp.full((16,), s * 3 + 1, i32)
            col2 = jnp.full((16,), s * 3 + 2, i32)
            rc0 = plsc.load_gather(rgb_v, [iota, col0])
            rc1 = plsc.load_gather(rgb_v, [iota, col1])
            rc2 = plsc.load_gather(rgb_v, [iota, col2])
            den = jnp.full((16,), 1.0, f32)
            s0 = jnp.zeros((16,), f32)
            s1 = jnp.zeros((16,), f32)
            s2 = jnp.zeros((16,), f32)
            for i in range(5):
                row = iota + i * G
                n0 = plsc.load_gather(nb_v, [row, col0])
                n1 = plsc.load_gather(nb_v, [row, col1])
                n2 = plsc.load_gather(nb_v, [row, col2])
                d0 = rc0 - n0
                d1 = rc1 - n1
                d2 = rc2 - n2
                wi = jnp.exp(-(d0 * d0 + d1 * d1 + d2 * d2))
                den = den + wi
                s0 = s0 + n0 * wi
                s1 = s1 + n1 * wi
                s2 = s2 + n2 * wi
            inv = 1.0 / den
            rn0 = jnp.where(m, (rc0 + s0) * inv, rc0)
            rn1 = jnp.where(m, (rc1 + s1) * inv, rc1)
            rn2 = jnp.where(m, (rc2 + s2) * inv, rc2)
            return (tr_next, zn, ar + w * rn0, ag + w * rn1, ab + w * rn2,
                    ad + w * zz, ao + w)

        zv16 = jnp.zeros((16,), f32)
        one16 = jnp.full((16,), 1.0, f32)
        z0 = plsc.load_gather(z_v, [iota, jnp.zeros((16,), i32)])
        _, _, ar, ag, ab, ad, ao = lax.fori_loop(
            0, N_SAMP, samp_body,
            (one16, z0, zv16, zv16, zv16, zv16, zv16))

        plsc.store_scatter(comp_s, [iota, jnp.zeros((16,), i32)], ar)
        plsc.store_scatter(comp_s, [iota, jnp.full((16,), 1, i32)], ag)
        plsc.store_scatter(comp_s, [iota, jnp.full((16,), 2, i32)], ab)
        d_s[...] = ad
        o_s[...] = ao
        pltpu.sync_copy(w_v.at[:, pl.ds(0, N_SAMP)], w_h.at[pl.ds(r0, G)])
        pltpu.sync_copy(comp_s, comp_h.at[pl.ds(r0, G)])
        pltpu.sync_copy(d_s, d_h.at[pl.ds(r0, G)])
        pltpu.sync_copy(o_s, o_h.at[pl.ds(r0, G)])

    bufs_a = (rgb_a, nb_a, sig_a, z_a)
    bufs_b = (rgb_b, nb_b, sig_b, z_b)
    fire(0, *bufs_a, sem_a)

    def pair_body(k, _):
        g0 = 2 * k
        drain(g0, *bufs_a, sem_a)
        fire(g0 + 1, *bufs_b, sem_b)
        compute(g0, *bufs_a)
        drain(g0 + 1, *bufs_b, sem_b)

        @pl.when(k < NG // 2 - 1)
        def _():
            fire(g0 + 2, *bufs_a, sem_a)

        compute(g0 + 1, *bufs_b)
        return 0

    lax.fori_loop(0, NG // 2, pair_body, 0)


@jax.jit
def _run_sc(rgb2, nbs2, sigma, z_vals):
    fn = functools.partial(
        pl.kernel,
        out_type=[
            jax.ShapeDtypeStruct((N_RAY, 3), f32),
            jax.ShapeDtypeStruct((N_RAY, N_SAMP), f32),
            jax.ShapeDtypeStruct((N_RAY,), f32),
            jax.ShapeDtypeStruct((N_RAY,), f32),
        ],
        mesh=plsc.VectorSubcoreMesh(core_axis_name="c", subcore_axis_name="s"),
        compiler_params=pltpu.CompilerParams(
            needs_layout_passes=False, use_tc_tiling_on_sc=False),
        scratch_types=[
            pltpu.VMEM((G, FLAT + 1), f32),
            pltpu.VMEM((5 * G, FLAT + 1), f32),
            pltpu.VMEM((G, N_SAMP + 1), f32),
            pltpu.VMEM((G, N_SAMP + 1), f32),
            pltpu.VMEM((G, FLAT + 1), f32),
            pltpu.VMEM((5 * G, FLAT + 1), f32),
            pltpu.VMEM((G, N_SAMP + 1), f32),
            pltpu.VMEM((G, N_SAMP + 1), f32),
            pltpu.VMEM((G, N_SAMP + 1), f32),
            pltpu.VMEM((G, 3), f32),
            pltpu.VMEM((16,), f32),
            pltpu.VMEM((16,), f32),
            pltpu.SemaphoreType.DMA,
            pltpu.SemaphoreType.DMA,
        ],
    )(_sc_body)
    return fn(rgb2, nbs2, sigma, z_vals)


def kernel(rgb, neighbor_rgbs, sigma, z_vals, white_bkgd):
    comp, w, depth, opac = _run_sc(
        rgb.reshape(N_RAY, FLAT), neighbor_rgbs.reshape(5 * N_RAY, FLAT),
        sigma, z_vals)
    comp_rgb = jnp.where(white_bkgd, comp + (1.0 - opac)[:, None], comp)
    return comp_rgb, depth, opac, w
